# baseline (speedup 1.0000x reference)
import jax
import jax.numpy as jnp
from jax import lax
from jax.experimental import pallas as pl
from jax.experimental.pallas import tpu as pltpu
from jax.experimental.pallas import tpu_sc as plsc

_K = 819
_N = 8192
_ROWS = 64
_NC, _NS, _L = 2, 16, 16
_NW = _NC * _NS
_RPW = _ROWS // _NW
_NB = 128
_HIST = _L * _NB
_CAND = _N + _L
_M31 = 0x7FFFFFFF
_UNROLL = 8
_GRP = _L * _UNROLL


def _sc_threshold_body(x_hbm, t_hbm, data_v, h0_v, h1_v, h2_v, h3_v,
                       c0_v, c1_v, tot0_v, tot1_v, tout_v, dma_sem):
    wid = lax.axis_index("s") * _NC + lax.axis_index("c")
    base = wid * _RPW
    cp = pltpu.make_async_copy(x_hbm.at[pl.ds(base, _RPW)], data_v, dma_sem)
    cp.start()

    lanes = lax.iota(jnp.int32, _L)
    zeros = jnp.zeros((_L,), jnp.int32)
    ones = jnp.ones((_L,), jnp.int32)
    lane15 = lanes == (_L - 1)
    cands = (c0_v, c1_v)
    tots = (tot0_v, tot1_v)

    def zero_body(i, c):
        for u in range(_UNROLL):
            h0_v[pl.ds(i * _GRP + u * _L, _L)] = zeros
            h1_v[pl.ds(i * _GRP + u * _L, _L)] = zeros
            h2_v[pl.ds(i * _GRP + u * _L, _L)] = zeros
            h3_v[pl.ds(i * _GRP + u * _L, _L)] = zeros
        return c
    lax.fori_loop(0, _HIST // _GRP, zero_body, 0)
    cp.wait()

    def load_group(i):
        bs = [[], []]
        for r in range(_RPW):
            for u in range(_UNROLL):
                v = data_v[r, pl.ds(i * _GRP + u * _L, _L)]
                bs[r].append(v & _M31)
        return bs

    def fold_hists(ha, hb):
        hs = (ha, hb)

        def body(i, c):
            for r in range(_RPW):
                for u in range(_UNROLL):
                    bkt = i * _UNROLL + u
                    v = hs[r][pl.ds(bkt * _L, _L)]
                    cv = plsc.cumsum(v)
                    plsc.store_scatter(tots[r],
                                       [jnp.broadcast_to(bkt, (_L,))],
                                       cv, mask=lane15)
            return c
        lax.fori_loop(0, _NB // _UNROLL, body, 0)

    def scan_tot(tot_v, rank):
        carry = jnp.int32(0)
        found = jnp.int32(0)
        sel = jnp.int32(0)
        r_in = jnp.int32(1)
        for c in range(_NB // _L - 1, -1, -1):
            rc = lax.rev(tot_v[pl.ds(c * _L, _L)], (0,))
            cum = plsc.cumsum(rc)
            rem = rank - carry
            crossed = cum >= rem
            npc = plsc.all_reduce_population_count(crossed)[0]
            above = jnp.max(jnp.where(crossed, jnp.int32(0), cum))
            upd = (found == 0) & (npc > 0)
            sel = jnp.where(upd, c * _L + npc - 1, sel)
            r_in = jnp.where(upd, rem - above, r_in)
            found = jnp.where(upd, jnp.int32(1), found)
            carry = carry + cum[_L - 1]
        return sel, r_in

    hists = (h0_v, h1_v)

    def pass1(i, c):
        bs = load_group(i)
        for r in range(_RPW):
            for u in range(_UNROLL):
                plsc.addupdate_scatter(hists[r],
                                       [(bs[r][u] >> 24) * _L + lanes],
                                       ones)
        return c
    lax.fori_loop(0, _N // _GRP, pass1, 0)
    fold_hists(h0_v, h1_v)
    sel0_0, r1_0 = scan_tot(tot0_v, jnp.int32(_K))
    sel0_1, r1_1 = scan_tot(tot1_v, jnp.int32(_K))
    sel0 = (sel0_0, sel0_1)

    hists2 = (h2_v, h3_v)

    def pass2(i, offs):
        bs = load_group(i)
        new_offs = []
        for r in range(_RPW):
            off = offs[r]
            for u in range(_UNROLL):
                b = bs[r][u]
                match = (b >> 24) == sel0[r]
                idx = ((b >> 17) & jnp.int32(0x7F)) * _L + lanes
                plsc.addupdate_scatter(hists2[r], [idx], ones, mask=match)
                mi = jnp.where(match, jnp.int32(1), jnp.int32(0))
                pos = off + plsc.cumsum(mi) - 1
                plsc.store_scatter(cands[r], [pos], b, mask=match)
                off = off + plsc.all_reduce_population_count(match)
            new_offs.append(off)
        return tuple(new_offs)
    offs1 = lax.fori_loop(0, _N // _GRP, pass2, (zeros, zeros))
    n1 = (offs1[0][0], offs1[1][0])
    fold_hists(h2_v, h3_v)
    sel1_0, r2_0 = scan_tot(tot0_v, r1_0)
    sel1_1, r2_1 = scan_tot(tot1_v, r1_1)
    pref = (sel0_0 * 128 + sel1_0, sel0_1 * 128 + sel1_1)
    r2 = (r2_0, r2_1)

    nvec1 = [(n1[r] + _L - 1) // _L for r in range(_RPW)]
    nvec1_max = jnp.maximum(nvec1[0], nvec1[1])
    prefs = [jnp.broadcast_to(pref[r], (_L,)) for r in range(_RPW)]

    def pass3(i, offs):
        new_offs = []
        for r in range(_RPW):
            off = offs[r]
            b = cands[r][pl.ds(i * _L, _L)]
            match = ((b >> 17) == prefs[r]) & (i * _L + lanes < n1[r])
            mi = jnp.where(match, jnp.int32(1), jnp.int32(0))
            pos = off + plsc.cumsum(mi) - 1
            plsc.store_scatter(cands[r], [pos], b, mask=match)
            new_offs.append(off + plsc.all_reduce_population_count(match))
        return tuple(new_offs)
    offs = lax.fori_loop(0, nvec1_max, pass3, (zeros, zeros))
    n2 = (offs[0][0], offs[1][0])
    c0_v[pl.ds(n2[0], _L)] = zeros
    c1_v[pl.ds(n2[1], _L)] = zeros
    nvec = [(n2[r] + _L - 1) // _L for r in range(_RPW)]
    nvec_max = jnp.maximum(nvec[0], nvec[1])

    r2s = [jnp.broadcast_to(r2[r], (_L,)) for r in range(_RPW)]
    pivs = [prefs[r] * 131072 for r in range(_RPW)]
    for bit in range(16, -1, -1):
        trials = [pivs[r] | (1 << bit) for r in range(_RPW)]

        def cnt_body(i, accs, trials=trials):
            a0, a1 = accs
            v0 = c0_v[pl.ds(i * _L, _L)]
            v1 = c1_v[pl.ds(i * _L, _L)]
            m0 = (v0 >= trials[0]) & (i < nvec[0])
            m1 = (v1 >= trials[1]) & (i < nvec[1])
            a0 = a0 + plsc.all_reduce_population_count(m0)
            a1 = a1 + plsc.all_reduce_population_count(m1)
            return (a0, a1)
        accs = lax.fori_loop(0, nvec_max, cnt_body, (zeros, zeros))
        for r in range(_RPW):
            pivs[r] = jnp.where(accs[r] >= r2s[r], trials[r], pivs[r])

    for r in range(_RPW):
        tout_v[r, :] = pivs[r]
    pltpu.sync_copy(tout_v, t_hbm.at[pl.ds(base, _RPW)])


_sc_threshold = pl.kernel(
    _sc_threshold_body,
    out_type=jax.ShapeDtypeStruct((_ROWS, _L), jnp.int32),
    mesh=plsc.VectorSubcoreMesh(
        core_axis_name="c", subcore_axis_name="s",
        num_cores=_NC, num_subcores=_NS,
    ),
    compiler_params=pltpu.CompilerParams(needs_layout_passes=False),
    scratch_types=[
        pltpu.VMEM((_RPW, _N), jnp.int32),
        pltpu.VMEM((_HIST,), jnp.int32),
        pltpu.VMEM((_HIST,), jnp.int32),
        pltpu.VMEM((_HIST,), jnp.int32),
        pltpu.VMEM((_HIST,), jnp.int32),
        pltpu.VMEM((_CAND,), jnp.int32),
        pltpu.VMEM((_CAND,), jnp.int32),
        pltpu.VMEM((_NB,), jnp.int32),
        pltpu.VMEM((_NB,), jnp.int32),
        pltpu.VMEM((_RPW, _L), jnp.int32),
        pltpu.SemaphoreType.DMA,
    ],
)


def _tc_finish_kernel(x_ref, t_ref, o_ref):
    x = x_ref[...]
    tb = t_ref[...][:, 0:1]
    t = lax.bitcast_convert_type(tb, jnp.float32)
    m = (jnp.abs(x) >= t).astype(jnp.float32)
    union = jnp.max(m, axis=0, keepdims=True)
    y = x * union
    s = jnp.sum(y * y, axis=1, keepdims=True)
    o_ref[...] = y / (jnp.sqrt(s) + 1e-6)


@jax.jit
def kernel(x):
    xbits = lax.bitcast_convert_type(x, jnp.int32)
    t = _sc_threshold(xbits)
    return pl.pallas_call(
        _tc_finish_kernel,
        out_shape=jax.ShapeDtypeStruct(x.shape, x.dtype),
    )(x, t)


# speedup vs baseline: 1.1811x; 1.1811x over previous
import jax
import jax.numpy as jnp
from jax import lax
from jax.experimental import pallas as pl
from jax.experimental.pallas import tpu as pltpu
from jax.experimental.pallas import tpu_sc as plsc

_K = 819
_N = 8192
_ROWS = 64
_NC, _NS, _L = 2, 16, 16
_NW = _NC * _NS
_RPW = _ROWS // _NW
_NB = 128
_HIST = _L * _NB
_CAND = _N + _L
_M31 = 0x7FFFFFFF
_UNROLL = 8
_GRP = _L * _UNROLL


def _sc_threshold_body(x_hbm, t_hbm, data_v, h0_v, h1_v, h2_v, h3_v,
                       c0_v, c1_v, tot0_v, tot1_v, tout_v, dma_sem):
    wid = lax.axis_index("s") * _NC + lax.axis_index("c")
    base = wid * _RPW
    cp = pltpu.make_async_copy(x_hbm.at[pl.ds(base, _RPW)], data_v, dma_sem)
    cp.start()

    lanes = lax.iota(jnp.int32, _L)
    zeros = jnp.zeros((_L,), jnp.int32)
    ones = jnp.ones((_L,), jnp.int32)
    lane15 = lanes == (_L - 1)
    cands = (c0_v, c1_v)
    tots = (tot0_v, tot1_v)

    def zero_body(i, c):
        for u in range(_UNROLL):
            h0_v[pl.ds(i * _GRP + u * _L, _L)] = zeros
            h1_v[pl.ds(i * _GRP + u * _L, _L)] = zeros
            h2_v[pl.ds(i * _GRP + u * _L, _L)] = zeros
            h3_v[pl.ds(i * _GRP + u * _L, _L)] = zeros
        return c
    lax.fori_loop(0, _HIST // _GRP, zero_body, 0)
    cp.wait()

    def load_group(i):
        bs = [[], []]
        for r in range(_RPW):
            for u in range(_UNROLL):
                v = data_v[r, pl.ds(i * _GRP + u * _L, _L)]
                bs[r].append(v & _M31)
        return bs

    def fold_hists(ha, hb):
        hs = (ha, hb)

        def body(i, c):
            for r in range(_RPW):
                for u in range(_UNROLL):
                    bkt = i * _UNROLL + u
                    v = hs[r][pl.ds(bkt * _L, _L)]
                    cv = plsc.cumsum(v)
                    plsc.store_scatter(tots[r],
                                       [jnp.broadcast_to(bkt, (_L,))],
                                       cv, mask=lane15)
            return c
        lax.fori_loop(0, _NB // _UNROLL, body, 0)

    def scan_tot(tot_v, rank):
        carry = jnp.int32(0)
        found = jnp.int32(0)
        sel = jnp.int32(0)
        r_in = jnp.int32(1)
        for c in range(_NB // _L - 1, -1, -1):
            rc = lax.rev(tot_v[pl.ds(c * _L, _L)], (0,))
            cum = plsc.cumsum(rc)
            rem = rank - carry
            crossed = cum >= rem
            npc = plsc.all_reduce_population_count(crossed)[0]
            above = jnp.max(jnp.where(crossed, jnp.int32(0), cum))
            upd = (found == 0) & (npc > 0)
            sel = jnp.where(upd, c * _L + npc - 1, sel)
            r_in = jnp.where(upd, rem - above, r_in)
            found = jnp.where(upd, jnp.int32(1), found)
            carry = carry + cum[_L - 1]
        return sel, r_in

    hists = (h0_v, h1_v)

    def pass1(i, c):
        bs = load_group(i)
        for r in range(_RPW):
            for u in range(_UNROLL):
                plsc.addupdate_scatter(hists[r],
                                       [(bs[r][u] >> 24) * _L + lanes],
                                       ones)
        return c
    lax.fori_loop(0, _N // _GRP, pass1, 0)
    fold_hists(h0_v, h1_v)
    sel0_0, r1_0 = scan_tot(tot0_v, jnp.int32(_K))
    sel0_1, r1_1 = scan_tot(tot1_v, jnp.int32(_K))
    sel0 = (sel0_0, sel0_1)

    hists2 = (h2_v, h3_v)

    def pass2(i, c):
        bs = load_group(i)
        for r in range(_RPW):
            for u in range(_UNROLL):
                b = bs[r][u]
                match = (b >> 24) == sel0[r]
                idx = ((b >> 17) & jnp.int32(0x7F)) * _L + lanes
                plsc.addupdate_scatter(hists2[r], [idx], ones, mask=match)
        return c
    lax.fori_loop(0, _N // _GRP, pass2, 0)
    fold_hists(h2_v, h3_v)
    sel1_0, r2_0 = scan_tot(tot0_v, r1_0)
    sel1_1, r2_1 = scan_tot(tot1_v, r1_1)
    pref = (sel0_0 * 128 + sel1_0, sel0_1 * 128 + sel1_1)
    r2 = (r2_0, r2_1)

    def pass3(i, offs):
        bs = load_group(i)
        new_offs = []
        for r in range(_RPW):
            off = offs[r]
            for u in range(_UNROLL):
                b = bs[r][u]
                match = (b >> 17) == pref[r]
                mi = jnp.where(match, jnp.int32(1), jnp.int32(0))
                pos = off + plsc.cumsum(mi) - 1
                plsc.store_scatter(cands[r], [pos], b, mask=match)
                off = off + plsc.all_reduce_population_count(match)
            new_offs.append(off)
        return tuple(new_offs)
    offs = lax.fori_loop(0, _N // _GRP, pass3, (zeros, zeros))
    n2 = (offs[0][0], offs[1][0])
    c0_v[pl.ds(n2[0], _L)] = zeros
    c1_v[pl.ds(n2[1], _L)] = zeros
    nvec = [(n2[r] + _L - 1) // _L for r in range(_RPW)]
    nvec_max = jnp.maximum(nvec[0], nvec[1])

    r2s = [jnp.broadcast_to(r2[r], (_L,)) for r in range(_RPW)]
    pivs = [jnp.broadcast_to(pref[r] * 131072, (_L,)) for r in range(_RPW)]
    for bit in range(16, -1, -1):
        trials = [pivs[r] | (1 << bit) for r in range(_RPW)]

        def cnt_body(i, accs, trials=trials):
            a0, a1 = accs
            v0 = c0_v[pl.ds(i * _L, _L)]
            v1 = c1_v[pl.ds(i * _L, _L)]
            m0 = (v0 >= trials[0]) & (i < nvec[0])
            m1 = (v1 >= trials[1]) & (i < nvec[1])
            a0 = a0 + plsc.all_reduce_population_count(m0)
            a1 = a1 + plsc.all_reduce_population_count(m1)
            return (a0, a1)
        accs = lax.fori_loop(0, nvec_max, cnt_body, (zeros, zeros))
        for r in range(_RPW):
            pivs[r] = jnp.where(accs[r] >= r2s[r], trials[r], pivs[r])

    for r in range(_RPW):
        tout_v[r, :] = pivs[r]
    pltpu.sync_copy(tout_v, t_hbm.at[pl.ds(base, _RPW)])


_sc_threshold = pl.kernel(
    _sc_threshold_body,
    out_type=jax.ShapeDtypeStruct((_ROWS, _L), jnp.int32),
    mesh=plsc.VectorSubcoreMesh(
        core_axis_name="c", subcore_axis_name="s",
        num_cores=_NC, num_subcores=_NS,
    ),
    compiler_params=pltpu.CompilerParams(needs_layout_passes=False),
    scratch_types=[
        pltpu.VMEM((_RPW, _N), jnp.int32),
        pltpu.VMEM((_HIST,), jnp.int32),
        pltpu.VMEM((_HIST,), jnp.int32),
        pltpu.VMEM((_HIST,), jnp.int32),
        pltpu.VMEM((_HIST,), jnp.int32),
        pltpu.VMEM((_CAND,), jnp.int32),
        pltpu.VMEM((_CAND,), jnp.int32),
        pltpu.VMEM((_NB,), jnp.int32),
        pltpu.VMEM((_NB,), jnp.int32),
        pltpu.VMEM((_RPW, _L), jnp.int32),
        pltpu.SemaphoreType.DMA,
    ],
)


def _tc_finish_kernel(x_ref, t_ref, o_ref):
    x = x_ref[...]
    tb = t_ref[...][:, 0:1]
    t = lax.bitcast_convert_type(tb, jnp.float32)
    m = (jnp.abs(x) >= t).astype(jnp.float32)
    union = jnp.max(m, axis=0, keepdims=True)
    y = x * union
    s = jnp.sum(y * y, axis=1, keepdims=True)
    o_ref[...] = y / (jnp.sqrt(s) + 1e-6)


@jax.jit
def kernel(x):
    xbits = lax.bitcast_convert_type(x, jnp.int32)
    t = _sc_threshold(xbits)
    return pl.pallas_call(
        _tc_finish_kernel,
        out_shape=jax.ShapeDtypeStruct(x.shape, x.dtype),
    )(x, t)
